# baseline (device time: 19749 ns/iter reference)
import jax
import jax.numpy as jnp
from jax import lax
from jax.experimental import pallas as pl
from jax.experimental.pallas import tpu as pltpu


def kernel(partial, resid, gamma):
    x = jnp.squeeze(partial, 0)
    g = gamma.reshape(1, -1)
    m, d = x.shape

    def body(x_ref, r_ref, g_ref, o_ref, comm_ref, send_sem, recv_sem):
        my_x = lax.axis_index("x")
        my_y = lax.axis_index("y")
        my_z = lax.axis_index("z")
        peer = (1 - my_x, my_y, my_z)

        barrier_sem = pltpu.get_barrier_semaphore()
        pl.semaphore_signal(
            barrier_sem, inc=1,
            device_id=peer, device_id_type=pl.DeviceIdType.MESH,
        )
        pl.semaphore_wait(barrier_sem, 1)

        rdma = pltpu.make_async_remote_copy(
            src_ref=x_ref,
            dst_ref=comm_ref,
            send_sem=send_sem,
            recv_sem=recv_sem,
            device_id=peer,
            device_id_type=pl.DeviceIdType.MESH,
        )
        rdma.start()
        rdma.wait()

        y = x_ref[:, :] + comm_ref[:, :] + r_ref[:, :]
        rms = jnp.sqrt(jnp.mean(y * y, axis=-1, keepdims=True) + 1e-6)
        o_ref[:, :] = y * (g_ref[:, :] / rms)

    return pl.pallas_call(
        body,
        out_shape=jax.ShapeDtypeStruct((m, d), jnp.float32),
        in_specs=[
            pl.BlockSpec(memory_space=pltpu.VMEM),
            pl.BlockSpec(memory_space=pltpu.VMEM),
            pl.BlockSpec(memory_space=pltpu.VMEM),
        ],
        out_specs=pl.BlockSpec(memory_space=pltpu.VMEM),
        scratch_shapes=[
            pltpu.VMEM((m, d), jnp.float32),
            pltpu.SemaphoreType.DMA,
            pltpu.SemaphoreType.DMA,
        ],
        compiler_params=pltpu.CompilerParams(collective_id=0),
    )(x, resid, g)


# device time: 18659 ns/iter; 1.0584x vs baseline; 1.0584x over previous
import jax
import jax.numpy as jnp
from jax import lax
from jax.experimental import pallas as pl
from jax.experimental.pallas import tpu as pltpu

QROWS = 128
C = 2
R = QROWS // C


def kernel(partial, resid, gamma):
    x = jnp.squeeze(partial, 0)
    g = gamma.reshape(1, -1)
    m, d = x.shape

    def body(x_ref, r_ref, g_ref, o_ref, rs_buf,
             rs_send, rs_recv, a1y_send, a1y_recv, a1z_send, a1z_recv,
             a2y_send, a2y_recv, a2z_send, a2z_recv):
        my_x = lax.axis_index("x")
        my_y = lax.axis_index("y")
        my_z = lax.axis_index("z")
        xpeer = (1 - my_x, my_y, my_z)
        ypeer = (my_x, 1 - my_y, my_z)
        zpeer = (my_x, my_y, 1 - my_z)

        q = 2 * my_y + my_z
        qy = 2 * (1 - my_y) + my_z
        qz = 2 * my_y + (1 - my_z)
        qd = 2 * (1 - my_y) + (1 - my_z)

        barrier_sem = pltpu.get_barrier_semaphore()
        for nbr in (xpeer, ypeer, zpeer):
            pl.semaphore_signal(
                barrier_sem, inc=1,
                device_id=nbr, device_id_type=pl.DeviceIdType.MESH,
            )
        pl.semaphore_wait(barrier_sem, 3)

        def rdma(src, dst, send_sem, recv_sem, peer):
            return pltpu.make_async_remote_copy(
                src_ref=src, dst_ref=dst,
                send_sem=send_sem, recv_sem=recv_sem,
                device_id=peer, device_id_type=pl.DeviceIdType.MESH,
            )

        rs = []
        for c in range(C):
            rows = pl.ds(q * QROWS + c * R, R)
            rs.append(rdma(x_ref.at[rows, :], rs_buf.at[pl.ds(c * R, R), :],
                           rs_send.at[c], rs_recv.at[c], xpeer))
            rs[c].start()

        a1y, a1z = [], []
        for c in range(C):
            rs[c].wait_recv()
            rows = pl.ds(q * QROWS + c * R, R)
            yv = x_ref[rows, :] + rs_buf[pl.ds(c * R, R), :] + r_ref[rows, :]
            rms = jnp.sqrt(jnp.mean(yv * yv, axis=-1, keepdims=True) + 1e-6)
            o_ref[rows, :] = yv * (g_ref[:, :] / rms)
            a1y.append(rdma(o_ref.at[rows, :], o_ref.at[rows, :],
                            a1y_send.at[c], a1y_recv.at[c], ypeer))
            a1z.append(rdma(o_ref.at[rows, :], o_ref.at[rows, :],
                            a1z_send.at[c], a1z_recv.at[c], zpeer))
            a1y[c].start()
            a1z[c].start()

        a1z[0].wait_recv()
        rows_zf = pl.ds(qz * QROWS, R)
        a2y = rdma(o_ref.at[rows_zf, :], o_ref.at[rows_zf, :],
                   a2y_send.at[0], a2y_recv.at[0], ypeer)
        a2y.start()

        a1y[1].wait_recv()
        rows_yf = pl.ds(qy * QROWS + R, R)
        a2z = rdma(o_ref.at[rows_yf, :], o_ref.at[rows_yf, :],
                   a2z_send.at[0], a2z_recv.at[0], zpeer)
        a2z.start()

        a1y[0].wait_recv()
        a1z[1].wait_recv()
        a2y.wait_recv()
        a2z.wait_recv()
        for c in range(C):
            rs[c].wait_send()
            a1y[c].wait_send()
            a1z[c].wait_send()
        a2y.wait_send()
        a2z.wait_send()

    return pl.pallas_call(
        body,
        out_shape=jax.ShapeDtypeStruct((m, d), jnp.float32),
        in_specs=[
            pl.BlockSpec(memory_space=pltpu.VMEM),
            pl.BlockSpec(memory_space=pltpu.VMEM),
            pl.BlockSpec(memory_space=pltpu.VMEM),
        ],
        out_specs=pl.BlockSpec(memory_space=pltpu.VMEM),
        scratch_shapes=[
            pltpu.VMEM((QROWS, d), jnp.float32),
            pltpu.SemaphoreType.DMA((C,)),
            pltpu.SemaphoreType.DMA((C,)),
            pltpu.SemaphoreType.DMA((C,)),
            pltpu.SemaphoreType.DMA((C,)),
            pltpu.SemaphoreType.DMA((C,)),
            pltpu.SemaphoreType.DMA((C,)),
            pltpu.SemaphoreType.DMA((1,)),
            pltpu.SemaphoreType.DMA((1,)),
            pltpu.SemaphoreType.DMA((1,)),
            pltpu.SemaphoreType.DMA((1,)),
        ],
        compiler_params=pltpu.CompilerParams(collective_id=0),
    )(x, resid, g)
